# baseline (speedup 1.0000x reference)
import functools

import jax
import jax.numpy as jnp
from jax.experimental import pallas as pl
from jax.experimental.pallas import tpu as pltpu

_EPS = 1e-8


def _normalize_body(a_ref, b_ref, o_ref, *, f):
    a = a_ref[...]
    b = b_ref[...]
    ss = (jnp.sum(a * a, axis=1, keepdims=True)
          + jnp.sum(b * b, axis=1, keepdims=True))
    inv = 1.0 / jnp.maximum(jnp.sqrt(ss), _EPS)
    o_ref[:, :f] = (a * inv).astype(o_ref.dtype)
    o_ref[:, f:] = (b * inv).astype(o_ref.dtype)


def _nn_body(x_ref, y_ref, sum_ref, acc_ref, *, bm, bn, cn, nj, d):
    i = pl.program_id(0)
    j = pl.program_id(1)

    @pl.when(j == 0)
    def _():
        acc_ref[...] = jnp.full_like(acc_ref, -3.0)

    xi = x_ref[...]

    def accum(mask_diag):
        m = None
        for c in range(bn // cn):
            yc = y_ref[c * cn:(c + 1) * cn, :]
            sub = jax.lax.dot_general(
                xi, yc, (((1,), (1,)), ((), ())),
                preferred_element_type=jnp.float32)
            if mask_diag:
                rows = jax.lax.broadcasted_iota(jnp.int32, (bm, cn), 0)
                cols = jax.lax.broadcasted_iota(jnp.int32, (bm, cn), 1)
                sub = jnp.where(rows == cols + (j * bn + c * cn - i * bm),
                                -1.0, sub)
            for l in range(cn // 128):
                piece = sub[:, l * 128:(l + 1) * 128]
                m = piece if m is None else jnp.maximum(m, piece)
        acc_ref[...] = jnp.maximum(acc_ref[...], m)

    on_diag = (i * bm) // bn == j
    pl.when(on_diag)(lambda: accum(True))
    pl.when(jnp.logical_not(on_diag))(lambda: accum(False))

    @pl.when(j == nj - 1)
    def _():
        mrow = jnp.max(acc_ref[...], axis=1, keepdims=True)
        d2 = jnp.maximum(2.0 - 2.0 * mrow, 0.0)
        dist = jnp.sqrt(d2 + d * (_EPS * _EPS))
        logd = jnp.log(dist + _EPS)
        sum_ref[0, 0, :] = jnp.full((128,), jnp.sum(logd), jnp.float32)


def kernel(emg_latent, emg_parallel_latent):
    n, f = emg_latent.shape
    d = 2 * f

    rb = min(n, 2048)
    xn = pl.pallas_call(
        functools.partial(_normalize_body, f=f),
        grid=(n // rb,),
        in_specs=[pl.BlockSpec((rb, f), lambda r: (r, 0)),
                  pl.BlockSpec((rb, f), lambda r: (r, 0))],
        out_specs=pl.BlockSpec((rb, d), lambda r: (r, 0)),
        out_shape=jax.ShapeDtypeStruct((n, d), jnp.bfloat16),
        compiler_params=pltpu.CompilerParams(
            dimension_semantics=("parallel",)),
        name="koleo_normalize",
    )(emg_latent, emg_parallel_latent)

    bm = min(n, 2048)
    bn = min(n, 8192)
    cn = min(bn, 256)
    ni = n // bm
    nj = n // bn

    sums = pl.pallas_call(
        functools.partial(_nn_body, bm=bm, bn=bn, cn=cn, nj=nj, d=d),
        grid=(ni, nj),
        in_specs=[pl.BlockSpec((bm, d), lambda i, j: (i, 0)),
                  pl.BlockSpec((bn, d), lambda i, j: (j, 0))],
        out_specs=pl.BlockSpec((1, 1, 128), lambda i, j: (i, 0, 0)),
        out_shape=jax.ShapeDtypeStruct((ni, 1, 128), jnp.float32),
        scratch_shapes=[pltpu.VMEM((bm, 128), jnp.float32)],
        compiler_params=pltpu.CompilerParams(
            dimension_semantics=("parallel", "arbitrary")),
        name="koleo_nn",
    )(xn, xn)

    return -(jnp.sum(sums[:, 0, 0]) / n)


# speedup vs baseline: 1.7955x; 1.7955x over previous
import functools

import jax
import jax.numpy as jnp
from jax.experimental import pallas as pl
from jax.experimental.pallas import tpu as pltpu

_EPS = 1e-8


def _normalize_body(a_ref, b_ref, o_ref, *, f):
    a = a_ref[...]
    b = b_ref[...]
    ss = (jnp.sum(a * a, axis=1, keepdims=True)
          + jnp.sum(b * b, axis=1, keepdims=True))
    inv = 1.0 / jnp.maximum(jnp.sqrt(ss), _EPS)
    o_ref[:, :f] = (a * inv).astype(o_ref.dtype)
    o_ref[:, f:] = (b * inv).astype(o_ref.dtype)


def _nn_body(x_ref, y_ref, sum_ref, acc_ref, *, bm, bn, cn, nj, d):
    i = pl.program_id(0)
    j = pl.program_id(1)

    @pl.when(j == 0)
    def _():
        acc_ref[...] = jnp.full_like(acc_ref, -3.0)

    xi = x_ref[...]

    def accum(mask_diag):
        m = None
        for c in range(bn // cn):
            yc = y_ref[c * cn:(c + 1) * cn, :]
            sub = jax.lax.dot_general(
                xi, yc, (((1,), (1,)), ((), ())),
                preferred_element_type=jnp.float32)
            if mask_diag:
                rows = jax.lax.broadcasted_iota(jnp.int32, (bm, cn), 0)
                cols = jax.lax.broadcasted_iota(jnp.int32, (bm, cn), 1)
                sub = jnp.where(rows == cols + (j * bn + c * cn - i * bm),
                                -1.0, sub)
            for l in range(cn // 128):
                piece = sub[:, l * 128:(l + 1) * 128]
                m = piece if m is None else jnp.maximum(m, piece)
        acc_ref[...] = jnp.maximum(acc_ref[...], m)

    on_diag = (i * bm) // bn == j
    pl.when(on_diag)(lambda: accum(True))
    pl.when(jnp.logical_not(on_diag))(lambda: accum(False))

    @pl.when(j == nj - 1)
    def _():
        mrow = jnp.max(acc_ref[...], axis=1, keepdims=True)
        d2 = jnp.maximum(2.0 - 2.0 * mrow, 0.0)
        dist = jnp.sqrt(d2 + d * (_EPS * _EPS))
        logd = jnp.log(dist + _EPS)
        sum_ref[0, 0, :] = jnp.full((128,), jnp.sum(logd), jnp.float32)


def kernel(emg_latent, emg_parallel_latent):
    n, f = emg_latent.shape
    d = 2 * f

    rb = min(n, 2048)
    xn = pl.pallas_call(
        functools.partial(_normalize_body, f=f),
        grid=(n // rb,),
        in_specs=[pl.BlockSpec((rb, f), lambda r: (r, 0)),
                  pl.BlockSpec((rb, f), lambda r: (r, 0))],
        out_specs=pl.BlockSpec((rb, d), lambda r: (r, 0)),
        out_shape=jax.ShapeDtypeStruct((n, d), jnp.bfloat16),
        compiler_params=pltpu.CompilerParams(
            dimension_semantics=("parallel",)),
        name="koleo_normalize",
    )(emg_latent, emg_parallel_latent)

    bm = min(n, 1024)
    bn = min(n, 8192)
    cn = min(bn, 512)
    ni = n // bm
    nj = n // bn

    sums = pl.pallas_call(
        functools.partial(_nn_body, bm=bm, bn=bn, cn=cn, nj=nj, d=d),
        grid=(ni, nj),
        in_specs=[pl.BlockSpec((bm, d), lambda i, j: (i, 0)),
                  pl.BlockSpec((bn, d), lambda i, j: (j, 0))],
        out_specs=pl.BlockSpec((1, 1, 128), lambda i, j: (i, 0, 0)),
        out_shape=jax.ShapeDtypeStruct((ni, 1, 128), jnp.float32),
        scratch_shapes=[pltpu.VMEM((bm, 128), jnp.float32)],
        compiler_params=pltpu.CompilerParams(
            dimension_semantics=("parallel", "arbitrary")),
        name="koleo_nn",
    )(xn, xn)

    return -(jnp.sum(sums[:, 0, 0]) / n)


# speedup vs baseline: 2.4677x; 1.3744x over previous
import functools

import jax
import jax.numpy as jnp
from jax.experimental import pallas as pl
from jax.experimental.pallas import tpu as pltpu

_EPS = 1e-8


def _normalize_body(a_ref, b_ref, o_ref, *, f):
    a = a_ref[...]
    b = b_ref[...]
    ss = (jnp.sum(a * a, axis=1, keepdims=True)
          + jnp.sum(b * b, axis=1, keepdims=True))
    inv = 1.0 / jnp.maximum(jnp.sqrt(ss), _EPS)
    o_ref[:, :f] = (a * inv).astype(o_ref.dtype)
    o_ref[:, f:] = (b * inv).astype(o_ref.dtype)


def _nn_tri_body(x_ref, y_ref, rowmax_ref, colmax_ref, *,
                 bm, bn, cn, njy, d):
    i = pl.program_id(0)
    jj = pl.program_id(1)
    r = bn // bm
    jblk = (i // r + jj) % njy

    @pl.when((i == 0) & (jj == 0))
    def _():
        colmax_ref[...] = jnp.full_like(colmax_ref, -3.0)

    xi = x_ref[...]

    def accum(mask_diag):
        m = None
        for c in range(bn // cn):
            yc = y_ref[c * cn:(c + 1) * cn, :]
            sub = jax.lax.dot_general(
                xi, yc, (((1,), (1,)), ((), ())),
                preferred_element_type=jnp.float32)
            if mask_diag:
                rows = jax.lax.broadcasted_iota(jnp.int32, (bm, cn), 0)
                cols = jax.lax.broadcasted_iota(jnp.int32, (bm, cn), 1)
                sub = jnp.where(
                    rows == cols + (jblk * bn + c * cn - i * bm), -1.0, sub)
            cf = jnp.max(sub.reshape(bm // 8, 8, cn), axis=0)
            slab = jblk * r + (c * cn) // bm
            off = (c * cn) % bm
            colmax_ref[slab, :, off:off + cn] = jnp.maximum(
                colmax_ref[slab, :, off:off + cn], cf)
            for l in range(cn // 128):
                piece = sub[:, l * 128:(l + 1) * 128]
                m = piece if m is None else jnp.maximum(m, piece)
        if mask_diag:
            rowmax_ref[0] = m
        else:
            rowmax_ref[0] = jnp.maximum(rowmax_ref[0], m)

    pl.when(jj == 0)(lambda: accum(True))
    pl.when(jj != 0)(lambda: accum(False))


def _final_body(rowmax_ref, colmax_ref, sum_ref, *, d):
    i = pl.program_id(0)
    mr = jnp.max(rowmax_ref[0], axis=1)
    cm = jnp.max(colmax_ref[i], axis=0)
    mrow = jnp.maximum(mr, cm)
    d2 = jnp.maximum(2.0 - 2.0 * mrow, 0.0)
    dist = jnp.sqrt(d2 + d * (_EPS * _EPS))
    logd = jnp.log(dist + _EPS)
    sum_ref[0, 0, :] = jnp.full((128,), jnp.sum(logd), jnp.float32)


def kernel(emg_latent, emg_parallel_latent):
    n, f = emg_latent.shape
    d = 2 * f

    rb = min(n, 2048)
    xn = pl.pallas_call(
        functools.partial(_normalize_body, f=f),
        grid=(n // rb,),
        in_specs=[pl.BlockSpec((rb, f), lambda q: (q, 0)),
                  pl.BlockSpec((rb, f), lambda q: (q, 0))],
        out_specs=pl.BlockSpec((rb, d), lambda q: (q, 0)),
        out_shape=jax.ShapeDtypeStruct((n, d), jnp.bfloat16),
        compiler_params=pltpu.CompilerParams(
            dimension_semantics=("parallel",)),
        name="koleo_normalize",
    )(emg_latent, emg_parallel_latent)

    bm = min(n, 1024)
    bn = min(n, 2048)
    cn = min(bn, 256)
    ni = n // bm
    njy = n // bn
    r = bn // bm
    nj2 = njy // 2 + 1

    rowmax, colmax = pl.pallas_call(
        functools.partial(_nn_tri_body, bm=bm, bn=bn, cn=cn, njy=njy, d=d),
        grid=(ni, nj2),
        in_specs=[
            pl.BlockSpec((bm, d), lambda i, jj: (i, 0)),
            pl.BlockSpec((bn, d),
                         lambda i, jj, r=r, njy=njy: ((i // r + jj) % njy, 0)),
        ],
        out_specs=[
            pl.BlockSpec((1, bm, 128), lambda i, jj: (i, 0, 0)),
            pl.BlockSpec((ni, 8, bm), lambda i, jj: (0, 0, 0)),
        ],
        out_shape=[
            jax.ShapeDtypeStruct((ni, bm, 128), jnp.float32),
            jax.ShapeDtypeStruct((ni, 8, bm), jnp.float32),
        ],
        compiler_params=pltpu.CompilerParams(
            dimension_semantics=("arbitrary", "arbitrary")),
        name="koleo_nn_tri",
    )(xn, xn)

    sums = pl.pallas_call(
        functools.partial(_final_body, d=d),
        grid=(ni,),
        in_specs=[pl.BlockSpec((1, bm, 128), lambda i: (i, 0, 0)),
                  pl.BlockSpec((ni, 8, bm), lambda i: (0, 0, 0))],
        out_specs=pl.BlockSpec((1, 1, 128), lambda i: (i, 0, 0)),
        out_shape=jax.ShapeDtypeStruct((ni, 1, 128), jnp.float32),
        compiler_params=pltpu.CompilerParams(
            dimension_semantics=("arbitrary",)),
        name="koleo_final",
    )(rowmax, colmax)

    return -(jnp.sum(sums[:, 0, 0]) / n)
